# Initial kernel scaffold; baseline (speedup 1.0000x reference)
import jax
import jax.numpy as jnp
from jax.experimental import pallas as pl


def kernel(x, adj, W_ih, W_hh, b_ih, b_hh, Wg, bg, Wm, W1, b1, W2, b2):
    raise NotImplementedError("write your pallas kernel here")


# speedup vs baseline: 11.1902x; 11.1902x over previous
import jax
import jax.numpy as jnp
from jax.experimental import pallas as pl
from jax.experimental.pallas import tpu as pltpu

_B = 512
_N = 16
_NVT = 16
_HS = 256
_NZ = 56
_VS = _HS + _N


def _dvae_body(xT_ref, adj_ref, wihT_ref, whhT_ref, bih_ref, bhh_ref,
               wgT_ref, bg_ref, wmT_ref, w1T_ref, b1_ref, w2T_ref, b2_ref,
               out_ref):
    Bb = xT_ref.shape[1]
    n = _N

    col = jax.lax.broadcasted_iota(jnp.int32, (1, n * n), 1)
    u_idx = col // n
    v_idx = col - u_idx * n
    tri = (u_idx < v_idx).astype(jnp.float32)
    maskf = adj_ref[...] * tri

    bih = bih_ref[...]
    bhh = bhh_ref[...]

    xx = xT_ref[...].reshape(n * Bb, _NVT)
    gi_all = jnp.dot(xx, wihT_ref[...], preferred_element_type=jnp.float32) + bih

    wgH = wgT_ref[: _HS, :]
    gb = wgT_ref[_HS:, :]
    wmH = wmT_ref[: _HS, :]
    mb = wmT_ref[_HS:, :]
    bg = bg_ref[...]
    whhT = whhT_ref[...]

    gated = []
    Hv = None
    for v in range(n):
        Hin = jnp.zeros((Bb, _HS), dtype=jnp.float32)
        for u in range(v):
            c = u * n + v
            Hin = Hin + maskf[:, c:c + 1] * gated[u]
        gh = jnp.dot(Hin, whhT, preferred_element_type=jnp.float32) + bhh
        gi = gi_all[v * Bb:(v + 1) * Bb, :]
        r = jax.nn.sigmoid(gi[:, :_HS] + gh[:, :_HS])
        z = jax.nn.sigmoid(gi[:, _HS:2 * _HS] + gh[:, _HS:2 * _HS])
        nn = jnp.tanh(gi[:, 2 * _HS:] + r * gh[:, 2 * _HS:])
        Hv = (1.0 - z) * nn + z * Hin
        if v < n - 1:
            g = jax.nn.sigmoid(
                jnp.dot(Hv, wgH, preferred_element_type=jnp.float32)
                + gb[v:v + 1, :] + bg)
            m = (jnp.dot(Hv, wmH, preferred_element_type=jnp.float32)
                 + mb[v:v + 1, :])
            gated.append(g * m)

    mu = jnp.dot(Hv, w1T_ref[...], preferred_element_type=jnp.float32) + b1_ref[...]
    lv = jnp.dot(Hv, w2T_ref[...], preferred_element_type=jnp.float32) + b2_ref[...]
    out_ref[0, :, :] = mu
    out_ref[1, :, :] = lv


def kernel(x, adj, W_ih, W_hh, b_ih, b_hh, Wg, bg, Wm, W1, b1, W2, b2):
    Bb = 256
    grid = (_B // Bb,)

    xT = jnp.transpose(x, (1, 0, 2))
    adjf = adj.astype(jnp.float32).reshape(_B, _N * _N)
    wihT = W_ih.T
    whhT = W_hh.T
    wgT = Wg.T
    wmT = Wm.T
    w1T = W1.T
    w2T = W2.T
    bih2 = b_ih.reshape(1, 3 * _HS)
    bhh2 = b_hh.reshape(1, 3 * _HS)
    bg2 = bg.reshape(1, _HS)
    b12 = b1.reshape(1, _NZ)
    b22 = b2.reshape(1, _NZ)

    out = pl.pallas_call(
        _dvae_body,
        grid=grid,
        in_specs=[
            pl.BlockSpec((_N, Bb, _NVT), lambda i: (0, i, 0)),
            pl.BlockSpec((Bb, _N * _N), lambda i: (i, 0)),
            pl.BlockSpec((_NVT, 3 * _HS), lambda i: (0, 0)),
            pl.BlockSpec((_HS, 3 * _HS), lambda i: (0, 0)),
            pl.BlockSpec((1, 3 * _HS), lambda i: (0, 0)),
            pl.BlockSpec((1, 3 * _HS), lambda i: (0, 0)),
            pl.BlockSpec((_VS, _HS), lambda i: (0, 0)),
            pl.BlockSpec((1, _HS), lambda i: (0, 0)),
            pl.BlockSpec((_VS, _HS), lambda i: (0, 0)),
            pl.BlockSpec((_HS, _NZ), lambda i: (0, 0)),
            pl.BlockSpec((1, _NZ), lambda i: (0, 0)),
            pl.BlockSpec((_HS, _NZ), lambda i: (0, 0)),
            pl.BlockSpec((1, _NZ), lambda i: (0, 0)),
        ],
        out_specs=pl.BlockSpec((2, Bb, _NZ), lambda i: (0, i, 0)),
        out_shape=jax.ShapeDtypeStruct((2, _B, _NZ), jnp.float32),
        compiler_params=pltpu.CompilerParams(
            dimension_semantics=("parallel",)),
    )(xT, adjf, wihT, whhT, bih2, bhh2, wgT, bg2, wmT, w1T, b12, w2T, b22)
    return out
